# Initial kernel scaffold; baseline (speedup 1.0000x reference)
import jax
import jax.numpy as jnp
from jax.experimental import pallas as pl


def kernel(x, W1, b1, W2, b2):
    raise NotImplementedError("write your pallas kernel here")


# speedup vs baseline: 195.4853x; 195.4853x over previous
import functools
import math

import jax
import jax.numpy as jnp
from jax.experimental import pallas as pl


def _gnn_body(x_ref, w1_ref, b1_ref, w2_ref, b2_ref, o_ref, *, n, alpha, beta, c_b, c_r):
    s, _, f = x_ref.shape
    h = w1_ref.shape[1]
    xs = x_ref[...]
    xw = jax.lax.dot_general(
        xs.reshape(s * n, f), w1_ref[...],
        (((1,), (0,)), ((), ())),
        preferred_element_type=jnp.float32,
    ).reshape(s, n, h)
    xb = xw[:, 0, :]
    s1 = jnp.sum(xw, axis=1) - xb
    b1 = b1_ref[...]
    t = jnp.maximum(b1[None, None, :] + alpha * xb[:, None, :] + 0.5 * xw, 0.0)
    S1 = jnp.sum(t, axis=1) - t[:, 0, :]
    hb = jnp.maximum(b1[None, :] + alpha * s1 + beta * xb, 0.0)
    v = c_b * hb + c_r * S1
    o_ref[...] = (b2_ref[...][None, :] + jnp.dot(
        v, w2_ref[...], preferred_element_type=jnp.float32))[None, ...]


def kernel(x, W1, b1, W2, b2):
    B, N, F = x.shape
    K1, H = W1.shape
    O = W2.shape[1]
    W1p = jnp.zeros((F, H), W1.dtype).at[:K1, :].set(W1)

    alpha = 1.0 / math.sqrt(2.0 * N)
    beta = 1.0 / N
    c_b = ((N - 1) * alpha + beta) / N
    c_r = (alpha + 0.5) / N

    S = 4
    assert B % S == 0
    body = functools.partial(_gnn_body, n=N, alpha=alpha, beta=beta,
                             c_b=c_b, c_r=c_r)
    return pl.pallas_call(
        body,
        grid=(B // S,),
        in_specs=[
            pl.BlockSpec((S, N, F), lambda i: (i, 0, 0)),
            pl.BlockSpec((F, H), lambda i: (0, 0)),
            pl.BlockSpec((H,), lambda i: (0,)),
            pl.BlockSpec((H, O), lambda i: (0, 0)),
            pl.BlockSpec((O,), lambda i: (0,)),
        ],
        out_specs=pl.BlockSpec((1, S, O), lambda i: (i, 0, 0)),
        out_shape=jax.ShapeDtypeStruct((B // S, S, O), x.dtype),
    )(x, W1p, b1, W2, b2).reshape(B, O)
